# baseline (speedup 1.0000x reference)
import functools

import jax
import jax.numpy as jnp
from jax import lax
from jax.experimental import pallas as pl
from jax.experimental.pallas import tpu as pltpu
from jax.experimental.pallas import tpu_sc as plsc

HID = 64
NC, NS, L = 2, 16, 16
NW = NC * NS
CH = 80


def _silu(v):
    return v * jax.nn.sigmoid(v)


def _node0_body(vx_ref, wvx_ref, b0_ref, w1a_ref, w1b_ref, b1_ref,
                h_ref, p_ref, q_ref):
    n = vx_ref.shape[0]
    h = _silu(jnp.dot(vx_ref[...], wvx_ref[...],
                      preferred_element_type=jnp.float32) + b0_ref[...])
    h_ref[pl.ds(0, n)] = h
    p_ref[pl.ds(0, n)] = jnp.dot(h, w1a_ref[...],
                                 preferred_element_type=jnp.float32) + b1_ref[...]
    q_ref[pl.ds(0, n)] = jnp.dot(h, w1b_ref[...],
                                 preferred_element_type=jnp.float32)
    z = jnp.zeros((h_ref.shape[0] - n, HID), jnp.float32)
    h_ref[pl.ds(n, h_ref.shape[0] - n)] = z
    p_ref[pl.ds(n, h_ref.shape[0] - n)] = z
    q_ref[pl.ds(n, h_ref.shape[0] - n)] = z


def _edge_body(m1_ref, w2_ref, b2_ref, out_ref):
    m = _silu(m1_ref[...])
    out_ref[...] = _silu(jnp.dot(m, w2_ref[...],
                                 preferred_element_type=jnp.float32)
                         + b2_ref[...])


def _node_body(h_ref, mia_ref, mib_ref, mw1a_ref, mw1b_ref, mb1_ref,
               mw2_ref, mb2_ref, w1a_ref, w1b_ref, b1_ref,
               h_out, p_out, q_out):
    n = h_ref.shape[0]
    m_i = (mia_ref[0, :n] + mia_ref[1, :n]
           + mib_ref[0, :n] + mib_ref[1, :n])
    t = _silu(jnp.dot(h_ref[...], mw1a_ref[...],
                      preferred_element_type=jnp.float32)
              + jnp.dot(m_i, mw1b_ref[...],
                        preferred_element_type=jnp.float32)
              + mb1_ref[...])
    h = _silu(jnp.dot(t, mw2_ref[...],
                      preferred_element_type=jnp.float32) + mb2_ref[...])
    h_out[...] = h
    p_out[...] = jnp.dot(h, w1a_ref[...],
                         preferred_element_type=jnp.float32) + b1_ref[...]
    q_out[...] = jnp.dot(h, w1b_ref[...], preferred_element_type=jnp.float32)


def _final_body(h_ref, mia_ref, mib_ref, mw1a_ref, mw1b_ref, mb1_ref,
                mw2_ref, mb2_ref, ow1_ref, ob1_ref, ow2_ref, ob2_ref,
                out_ref):
    n = h_ref.shape[0]
    m_i = (mia_ref[0, :n] + mia_ref[1, :n]
           + mib_ref[0, :n] + mib_ref[1, :n])
    t = _silu(jnp.dot(h_ref[...], mw1a_ref[...],
                      preferred_element_type=jnp.float32)
              + jnp.dot(m_i, mw1b_ref[...],
                        preferred_element_type=jnp.float32)
              + mb1_ref[...])
    h = _silu(jnp.dot(t, mw2_ref[...],
                      preferred_element_type=jnp.float32) + mb2_ref[...])
    o = _silu(jnp.dot(h, ow1_ref[...],
                      preferred_element_type=jnp.float32) + ob1_ref[...])
    out_ref[...] = jnp.dot(o, ow2_ref[...],
                           preferred_element_type=jnp.float32) + ob2_ref[...]


def _make_ea_kernel(E, N):
    EW = E // NW
    NCH = EW // CH
    mesh = plsc.VectorSubcoreMesh(core_axis_name="c", subcore_axis_name="s")

    @functools.partial(
        pl.kernel, mesh=mesh,
        compiler_params=pltpu.CompilerParams(needs_layout_passes=False, use_tc_tiling_on_sc=False),
        out_type=jax.ShapeDtypeStruct((NW, NCH, CH), jnp.float32),
        scratch_types=[
            pltpu.VMEM((N,), jnp.float32),
            pltpu.VMEM((NCH, CH), jnp.int32),
            pltpu.VMEM((NCH, CH), jnp.int32),
            pltpu.VMEM((NCH, CH), jnp.float32),
        ],
    )
    def ea_kernel(ch_hbm, row_hbm, col_hbm, out_hbm, chv, rv, cv, ov):
        wid = lax.axis_index("s") * NC + lax.axis_index("c")
        pltpu.sync_copy(ch_hbm, chv)
        pltpu.sync_copy(row_hbm.at[wid], rv)
        pltpu.sync_copy(col_hbm.at[wid], cv)

        @plsc.parallel_loop(0, NCH, unroll=2)
        def chunk(j):
            for t in range(CH // L):
                sl = pl.ds(t * L, L)
                ir = rv[j, sl]
                ic = cv[j, sl]
                ov[j, sl] = (plsc.load_gather(chv, [ir])
                             * plsc.load_gather(chv, [ic]))
        pltpu.sync_copy(ov, out_hbm.at[wid])

    return ea_kernel


def _make_gather_kernel(NCH, N, NPAD):
    EW = NCH * CH
    E = EW * NW
    NRT = NPAD // NS
    NBUF = 3
    mesh = plsc.VectorSubcoreMesh(core_axis_name="c", subcore_axis_name="s")

    @functools.partial(
        pl.kernel, mesh=mesh,
        compiler_params=pltpu.CompilerParams(needs_layout_passes=False, use_tc_tiling_on_sc=False),
        out_type=jax.ShapeDtypeStruct((E // 2, 2 * HID), jnp.float32),
        scratch_types=(
            [pltpu.VMEM((NCH, CH), jnp.int32),
             pltpu.VMEM((NCH, CH), jnp.int32),
             pltpu.VMEM((NCH, CH), jnp.float32)]
            + [pltpu.VMEM((CH, HID), jnp.float32)] * (2 * NBUF)
            + [pltpu.VMEM((CH // 2, 2 * HID), jnp.float32)] * NBUF
            + [pltpu.VMEM((HID,), jnp.float32),
               pltpu.VMEM_SHARED((NPAD, HID), jnp.float32)]
            + [pltpu.SemaphoreType.DMA] * (3 * NBUF)
        ),
    )
    def gather_kernel(p_hbm, q_hbm, row_hbm, col_hbm, ea_hbm, wea_hbm,
                      out_hbm, rv, cv, eav, *bufs):
        pbs = list(bufs[0:NBUF])
        qbs = list(bufs[NBUF:2 * NBUF])
        obs = list(bufs[2 * NBUF:3 * NBUF])
        wv = bufs[3 * NBUF]
        psp = bufs[3 * NBUF + 1]
        semps = list(bufs[3 * NBUF + 2:3 * NBUF + 2 + NBUF])
        semqs = list(bufs[3 * NBUF + 2 + NBUF:3 * NBUF + 2 + 2 * NBUF])
        semos = list(bufs[3 * NBUF + 2 + 2 * NBUF:])
        s = lax.axis_index("s")
        wid = s * NC + lax.axis_index("c")
        pltpu.sync_copy(p_hbm.at[pl.ds(s * NRT, NRT)],
                        psp.at[pl.ds(s * NRT, NRT)])
        pltpu.sync_copy(row_hbm.at[wid], rv)
        pltpu.sync_copy(col_hbm.at[wid], cv)
        pltpu.sync_copy(ea_hbm.at[wid], eav)
        pltpu.sync_copy(wea_hbm, wv)
        wvs = [wv[pl.ds(t * L, L)] for t in range(HID // L)]
        plsc.subcore_barrier()

        for jp in range(2):
            pltpu.async_copy(psp.at[rv.at[jp]], pbs[jp], semps[jp])
            pltpu.async_copy(q_hbm.at[cv.at[jp]], qbs[jp], semqs[jp])

        def chunk(j, carry):
            def body(b):
                pb, qb, ob = pbs[b], qbs[b], obs[b]

                @pl.when(j < NCH - 2)
                def _issue():
                    nb = (b + 2) % NBUF
                    pltpu.async_copy(psp.at[rv.at[j + 2]], pbs[nb],
                                     semps[nb])
                    pltpu.async_copy(q_hbm.at[cv.at[j + 2]], qbs[nb],
                                     semqs[nb])

                pltpu.make_async_copy(psp.at[rv.at[j]], pb, semps[b]).wait()
                pltpu.make_async_copy(q_hbm.at[cv.at[j]], qb, semqs[b]).wait()

                @pl.when(j >= NBUF)
                def _drain():
                    pltpu.make_async_copy(
                        ob, out_hbm.at[pl.ds(wid * EW // 2, CH // 2)],
                        semos[b]).wait()

                jv = jnp.full((L,), j, jnp.int32)

                @plsc.parallel_loop(0, CH // 2, unroll=2)
                def edge(i2):
                    for half in range(2):
                        i = 2 * i2 + half
                        iv = jnp.full((L,), i, jnp.int32)
                        ev = plsc.load_gather(eav, [jv, iv])
                        for t in range(HID // L):
                            ob[i2, pl.ds(half * HID + t * L, L)] = (
                                pb[i, pl.ds(t * L, L)]
                                + qb[i, pl.ds(t * L, L)] + ev * wvs[t])
                pltpu.async_copy(
                    ob, out_hbm.at[pl.ds((wid * EW + j * CH) // 2, CH // 2)],
                    semos[b])

            for b in range(NBUF):
                @pl.when(j % NBUF == b)
                def _b(b=b):
                    body(b)

            return carry

        lax.fori_loop(0, NCH, chunk, 0)
        for b in range(NBUF):
            pltpu.make_async_copy(
                obs[b], out_hbm.at[pl.ds(wid * EW // 2, CH // 2)],
                semos[b]).wait()

    return gather_kernel


def _make_scatter_kernel(NCH, N, NPAD):
    EW = NCH * CH
    E = EW * NW
    NRT = NPAD // NS
    mesh = plsc.VectorSubcoreMesh(core_axis_name="c", subcore_axis_name="s")

    @functools.partial(
        pl.kernel, mesh=mesh,
        compiler_params=pltpu.CompilerParams(needs_layout_passes=False, use_tc_tiling_on_sc=False),
        out_type=jax.ShapeDtypeStruct((NC, NPAD, HID), jnp.float32),
        scratch_types=[
            pltpu.VMEM((NCH, CH), jnp.int32),
            pltpu.VMEM((CH // 2, 2 * HID), jnp.float32),
            pltpu.VMEM((CH // 2, 2 * HID), jnp.float32),
            pltpu.VMEM((CH, HID), jnp.float32),
            pltpu.VMEM((CH, HID), jnp.float32),
            pltpu.VMEM_SHARED((NPAD, HID), jnp.float32),
            pltpu.SemaphoreType.DMA,
            pltpu.SemaphoreType.DMA,
            pltpu.SemaphoreType.DMA,
            pltpu.SemaphoreType.DMA,
        ],
    )
    def scatter_kernel(m2_hbm, row_hbm, zeros_hbm, out_hbm, rv, sb0, sb1,
                       mb0, mb1, acc, sems0, sems1, sema0, sema1):
        c = lax.axis_index("c")
        s = lax.axis_index("s")
        wid = s * NC + c
        sbs, mbs = [sb0, sb1], [mb0, mb1]
        semss, semas = [sems0, sems1], [sema0, sema1]
        pltpu.sync_copy(row_hbm.at[wid], rv)
        pltpu.async_copy(m2_hbm.at[pl.ds(wid * EW // 2, CH // 2)], sb0, sems0)
        pltpu.sync_copy(zeros_hbm.at[pl.ds(s * NRT, NRT)],
                        acc.at[pl.ds(s * NRT, NRT)])
        plsc.subcore_barrier()

        def chunk(j, carry):
            def body(b):
                nb = 1 - b

                @pl.when(j < NCH - 1)
                def _issue():
                    pltpu.async_copy(
                        m2_hbm.at[pl.ds((wid * EW + (j + 1) * CH) // 2,
                                        CH // 2)],
                        sbs[nb], semss[nb])

                pltpu.make_async_copy(
                    m2_hbm.at[pl.ds(wid * EW // 2, CH // 2)],
                    sbs[b], semss[b]).wait()

                @pl.when(j >= 2)
                def _drain():
                    pltpu.make_async_copy(
                        mbs[b], acc.at[rv.at[j]], semas[b]).wait()

                @plsc.parallel_loop(0, CH // 2, unroll=2)
                def repack(i2):
                    for half in range(2):
                        for t in range(HID // L):
                            mbs[b][2 * i2 + half, pl.ds(t * L, L)] = (
                                sbs[b][i2, pl.ds(half * HID + t * L, L)])
                pltpu.async_copy(mbs[b], acc.at[rv.at[j]], semas[b],
                                 add=True)

            @pl.when(j % 2 == 0)
            def _b0():
                body(0)

            @pl.when(j % 2 == 1)
            def _b1():
                body(1)

            return carry

        lax.fori_loop(0, NCH, chunk, 0)
        for b in range(2):
            pltpu.make_async_copy(mbs[b], acc.at[rv.at[0]], semas[b]).wait()
        plsc.subcore_barrier()
        pltpu.sync_copy(acc.at[pl.ds(s * NRT, NRT)],
                        out_hbm.at[c, pl.ds(s * NRT, NRT)])

    return scatter_kernel


def kernel(vel_norms, x, edge_index, charges, params):
    N = vel_norms.shape[0]
    E = edge_index.shape[1]
    LAYERS = 4

    f32 = jnp.float32

    w0a = params['node0_W'][:-3]
    w0x = params['node0_W'][-3:]
    wv = params['venc_W'] @ w0a
    b0 = (params['venc_b'] @ w0a + params['node0_b'])[None, :]
    wvx = jnp.concatenate([wv, w0x], axis=0)

    w1a = [params['edge%d_W1' % i][:HID] for i in range(LAYERS)]
    w1b = [params['edge%d_W1' % i][HID:2 * HID] for i in range(LAYERS)]
    wea = [params['edge%d_W1' % i][2 * HID] for i in range(LAYERS)]
    b1 = [params['edge%d_b1' % i][None, :] for i in range(LAYERS)]
    w2 = [params['edge%d_W2' % i] for i in range(LAYERS)]
    b2 = [params['edge%d_b2' % i][None, :] for i in range(LAYERS)]
    mw1a = [params['msg%d_W1' % i][:HID] for i in range(LAYERS)]
    mw1b = [params['msg%d_W1' % i][HID:] for i in range(LAYERS)]
    mb1 = [params['msg%d_b1' % i][None, :] for i in range(LAYERS)]
    mw2 = [params['msg%d_W2' % i] for i in range(LAYERS)]
    mb2 = [params['msg%d_b2' % i][None, :] for i in range(LAYERS)]

    OPAD = 128
    ow2p = jnp.zeros((HID, OPAD), f32).at[:, :3].set(params['out_W2'])
    ob2p = jnp.zeros((1, OPAD), f32).at[:, :3].set(params['out_b2'])
    ob1 = params['out_b1'][None, :]

    NCH = E // (NW * CH)
    row2 = edge_index[0].reshape(NW, NCH, CH)
    col2 = edge_index[1].reshape(NW, NCH, CH)
    NPAD = ((N + 8 * NS - 1) // (8 * NS)) * (8 * NS)
    zeros_n = jnp.zeros((NPAD, HID), f32)
    vx = jnp.concatenate([vel_norms, x], axis=1)

    def tc_call(body, out_shape, *args):
        return pl.pallas_call(body, out_shape=out_shape)(*args)

    nspec = jax.ShapeDtypeStruct((NPAD, HID), f32)
    h, p_tab, q_tab = tc_call(
        _node0_body, (nspec, nspec, nspec),
        vx, wvx, b0, w1a[0], w1b[0], b1[0])

    ea_kernel = _make_ea_kernel(E, N)
    NCH_A = NCH // 2
    NCH_B = NCH - NCH_A
    gather_a = _make_gather_kernel(NCH_A, N, NPAD)
    gather_b = _make_gather_kernel(NCH_B, N, NPAD)
    scatter_a = _make_scatter_kernel(NCH_A, N, NPAD)
    scatter_b = _make_scatter_kernel(NCH_B, N, NPAD)

    ea2 = ea_kernel(charges, row2, col2)
    row_a, row_b = row2[:, :NCH_A], row2[:, NCH_A:]
    col_a, col_b = col2[:, :NCH_A], col2[:, NCH_A:]
    ea_a, ea_b = ea2[:, :NCH_A], ea2[:, NCH_A:]

    H2 = 2 * HID
    w2d = [jnp.zeros((H2, H2), f32).at[:HID, :HID].set(w)
           .at[HID:, HID:].set(w) for w in w2]
    b2d = [jnp.concatenate([b, b], axis=1) for b in b2]

    def make_edge_call(nch):
        rows = nch * CH * NW // 2
        eb = rows // 8
        return pl.pallas_call(
            _edge_body,
            grid=(rows // eb,),
            in_specs=[
                pl.BlockSpec((eb, H2), lambda i: (i, 0)),
                pl.BlockSpec((H2, H2), lambda i: (0, 0)),
                pl.BlockSpec((1, H2), lambda i: (0, 0)),
            ],
            out_specs=pl.BlockSpec((eb, H2), lambda i: (i, 0)),
            out_shape=jax.ShapeDtypeStruct((rows, H2), f32),
        )

    edge_call_a = make_edge_call(NCH_A)
    edge_call_b = make_edge_call(NCH_B)

    for i in range(LAYERS):
        m1a = gather_a(p_tab, q_tab, row_a, col_a, ea_a, wea[i])
        m1b = gather_b(p_tab, q_tab, row_b, col_b, ea_b, wea[i])
        m2a = edge_call_a(m1a, w2d[i], b2d[i])
        m2b = edge_call_b(m1b, w2d[i], b2d[i])
        mia = scatter_a(m2a, row_a, zeros_n)
        mib = scatter_b(m2b, row_b, zeros_n)
        if i < LAYERS - 1:
            h, p_tab, q_tab = tc_call(
                _node_body, (nspec, nspec, nspec),
                h, mia, mib, mw1a[i], mw1b[i], mb1[i], mw2[i], mb2[i],
                w1a[i + 1], w1b[i + 1], b1[i + 1])
        else:
            pred = tc_call(
                _final_body, jax.ShapeDtypeStruct((NPAD, OPAD), f32),
                h, mia, mib, mw1a[i], mw1b[i], mb1[i], mw2[i], mb2[i],
                params['out_W1'], ob1, ow2p, ob2p)

    return pred[:N, :3]


# speedup vs baseline: 1.0354x; 1.0354x over previous
import functools

import jax
import jax.numpy as jnp
from jax import lax
from jax.experimental import pallas as pl
from jax.experimental.pallas import tpu as pltpu
from jax.experimental.pallas import tpu_sc as plsc

HID = 64
NC, NS, L = 2, 16, 16
NW = NC * NS
CH = 80


def _silu(v):
    return v * jax.nn.sigmoid(v)


def _node0_body(vx_ref, wvx_ref, b0_ref, w1a_ref, w1b_ref, b1_ref,
                h_ref, p_ref, q_ref):
    n = vx_ref.shape[0]
    h = _silu(jnp.dot(vx_ref[...], wvx_ref[...],
                      preferred_element_type=jnp.float32) + b0_ref[...])
    h_ref[pl.ds(0, n)] = h
    p_ref[pl.ds(0, n)] = jnp.dot(h, w1a_ref[...],
                                 preferred_element_type=jnp.float32) + b1_ref[...]
    q_ref[pl.ds(0, n)] = jnp.dot(h, w1b_ref[...],
                                 preferred_element_type=jnp.float32)
    z = jnp.zeros((h_ref.shape[0] - n, HID), jnp.float32)
    h_ref[pl.ds(n, h_ref.shape[0] - n)] = z
    p_ref[pl.ds(n, h_ref.shape[0] - n)] = z
    q_ref[pl.ds(n, h_ref.shape[0] - n)] = z


def _edge_body(m1_ref, w2_ref, b2_ref, out_ref):
    m = _silu(m1_ref[...])
    out_ref[...] = _silu(jnp.dot(m, w2_ref[...],
                                 preferred_element_type=jnp.float32)
                         + b2_ref[...])


def _node_body(h_ref, mi_ref, mw1a_ref, mw1b_ref, mb1_ref,
               mw2_ref, mb2_ref, w1a_ref, w1b_ref, b1_ref,
               h_out, p_out, q_out):
    n = h_ref.shape[0]
    m_i = mi_ref[0, :n] + mi_ref[1, :n]
    t = _silu(jnp.dot(h_ref[...], mw1a_ref[...],
                      preferred_element_type=jnp.float32)
              + jnp.dot(m_i, mw1b_ref[...],
                        preferred_element_type=jnp.float32)
              + mb1_ref[...])
    h = _silu(jnp.dot(t, mw2_ref[...],
                      preferred_element_type=jnp.float32) + mb2_ref[...])
    h_out[...] = h
    p_out[...] = jnp.dot(h, w1a_ref[...],
                         preferred_element_type=jnp.float32) + b1_ref[...]
    q_out[...] = jnp.dot(h, w1b_ref[...], preferred_element_type=jnp.float32)


def _final_body(h_ref, mi_ref, mw1a_ref, mw1b_ref, mb1_ref,
                mw2_ref, mb2_ref, ow1_ref, ob1_ref, ow2_ref, ob2_ref,
                out_ref):
    n = h_ref.shape[0]
    m_i = mi_ref[0, :n] + mi_ref[1, :n]
    t = _silu(jnp.dot(h_ref[...], mw1a_ref[...],
                      preferred_element_type=jnp.float32)
              + jnp.dot(m_i, mw1b_ref[...],
                        preferred_element_type=jnp.float32)
              + mb1_ref[...])
    h = _silu(jnp.dot(t, mw2_ref[...],
                      preferred_element_type=jnp.float32) + mb2_ref[...])
    o = _silu(jnp.dot(h, ow1_ref[...],
                      preferred_element_type=jnp.float32) + ob1_ref[...])
    out_ref[...] = jnp.dot(o, ow2_ref[...],
                           preferred_element_type=jnp.float32) + ob2_ref[...]


def _make_ea_kernel(E, N):
    EW = E // NW
    NCH = EW // CH
    mesh = plsc.VectorSubcoreMesh(core_axis_name="c", subcore_axis_name="s")

    @functools.partial(
        pl.kernel, mesh=mesh,
        compiler_params=pltpu.CompilerParams(needs_layout_passes=False, use_tc_tiling_on_sc=False),
        out_type=jax.ShapeDtypeStruct((NW, NCH, CH), jnp.float32),
        scratch_types=[
            pltpu.VMEM((N,), jnp.float32),
            pltpu.VMEM((NCH, CH), jnp.int32),
            pltpu.VMEM((NCH, CH), jnp.int32),
            pltpu.VMEM((NCH, CH), jnp.float32),
        ],
    )
    def ea_kernel(ch_hbm, row_hbm, col_hbm, out_hbm, chv, rv, cv, ov):
        wid = lax.axis_index("s") * NC + lax.axis_index("c")
        pltpu.sync_copy(ch_hbm, chv)
        pltpu.sync_copy(row_hbm.at[wid], rv)
        pltpu.sync_copy(col_hbm.at[wid], cv)

        @plsc.parallel_loop(0, NCH, unroll=2)
        def chunk(j):
            for t in range(CH // L):
                sl = pl.ds(t * L, L)
                ir = rv[j, sl]
                ic = cv[j, sl]
                ov[j, sl] = (plsc.load_gather(chv, [ir])
                             * plsc.load_gather(chv, [ic]))
        pltpu.sync_copy(ov, out_hbm.at[wid])

    return ea_kernel


def _make_gather_kernel(NCH, N, NPAD):
    EW = NCH * CH
    E = EW * NW
    NRT = NPAD // NS
    NBUF = 3
    mesh = plsc.VectorSubcoreMesh(core_axis_name="c", subcore_axis_name="s")

    @functools.partial(
        pl.kernel, mesh=mesh,
        compiler_params=pltpu.CompilerParams(needs_layout_passes=False, use_tc_tiling_on_sc=False),
        out_type=jax.ShapeDtypeStruct((E // 2, 2 * HID), jnp.float32),
        scratch_types=(
            [pltpu.VMEM((NCH, CH), jnp.int32),
             pltpu.VMEM((NCH, CH), jnp.int32),
             pltpu.VMEM((NCH, CH), jnp.float32)]
            + [pltpu.VMEM((CH, HID), jnp.float32)] * (2 * NBUF)
            + [pltpu.VMEM((CH // 2, 2 * HID), jnp.float32)] * NBUF
            + [pltpu.VMEM((HID,), jnp.float32),
               pltpu.VMEM_SHARED((NPAD, HID), jnp.float32)]
            + [pltpu.SemaphoreType.DMA] * (3 * NBUF)
        ),
    )
    def gather_kernel(p_hbm, q_hbm, row_hbm, col_hbm, ea_hbm, wea_hbm,
                      out_hbm, rv, cv, eav, *bufs):
        pbs = list(bufs[0:NBUF])
        qbs = list(bufs[NBUF:2 * NBUF])
        obs = list(bufs[2 * NBUF:3 * NBUF])
        wv = bufs[3 * NBUF]
        psp = bufs[3 * NBUF + 1]
        semps = list(bufs[3 * NBUF + 2:3 * NBUF + 2 + NBUF])
        semqs = list(bufs[3 * NBUF + 2 + NBUF:3 * NBUF + 2 + 2 * NBUF])
        semos = list(bufs[3 * NBUF + 2 + 2 * NBUF:])
        s = lax.axis_index("s")
        wid = s * NC + lax.axis_index("c")
        pltpu.sync_copy(p_hbm.at[pl.ds(s * NRT, NRT)],
                        psp.at[pl.ds(s * NRT, NRT)])
        pltpu.sync_copy(row_hbm.at[wid], rv)
        pltpu.sync_copy(col_hbm.at[wid], cv)
        pltpu.sync_copy(ea_hbm.at[wid], eav)
        pltpu.sync_copy(wea_hbm, wv)
        wvs = [wv[pl.ds(t * L, L)] for t in range(HID // L)]
        plsc.subcore_barrier()

        for jp in range(2):
            pltpu.async_copy(psp.at[rv.at[jp]], pbs[jp], semps[jp])
            pltpu.async_copy(q_hbm.at[cv.at[jp]], qbs[jp], semqs[jp])

        def chunk(j, carry):
            def body(b):
                pb, qb, ob = pbs[b], qbs[b], obs[b]

                @pl.when(j < NCH - 2)
                def _issue():
                    nb = (b + 2) % NBUF
                    pltpu.async_copy(psp.at[rv.at[j + 2]], pbs[nb],
                                     semps[nb])
                    pltpu.async_copy(q_hbm.at[cv.at[j + 2]], qbs[nb],
                                     semqs[nb])

                pltpu.make_async_copy(psp.at[rv.at[j]], pb, semps[b]).wait()
                pltpu.make_async_copy(q_hbm.at[cv.at[j]], qb, semqs[b]).wait()

                @pl.when(j >= NBUF)
                def _drain():
                    pltpu.make_async_copy(
                        ob, out_hbm.at[pl.ds(wid * EW // 2, CH // 2)],
                        semos[b]).wait()

                jv = jnp.full((L,), j, jnp.int32)

                @plsc.parallel_loop(0, CH // 2, unroll=2)
                def edge(i2):
                    for half in range(2):
                        i = 2 * i2 + half
                        iv = jnp.full((L,), i, jnp.int32)
                        ev = plsc.load_gather(eav, [jv, iv])
                        for t in range(HID // L):
                            ob[i2, pl.ds(half * HID + t * L, L)] = (
                                pb[i, pl.ds(t * L, L)]
                                + qb[i, pl.ds(t * L, L)] + ev * wvs[t])
                pltpu.async_copy(
                    ob, out_hbm.at[pl.ds((wid * EW + j * CH) // 2, CH // 2)],
                    semos[b])

            for b in range(NBUF):
                @pl.when(j % NBUF == b)
                def _b(b=b):
                    body(b)

            return carry

        lax.fori_loop(0, NCH, chunk, 0)
        for b in range(NBUF):
            pltpu.make_async_copy(
                obs[b], out_hbm.at[pl.ds(wid * EW // 2, CH // 2)],
                semos[b]).wait()

    return gather_kernel


def _make_scatter_kernel(NCH, N, NPAD):
    EW = NCH * CH
    E = EW * NW
    NRT = NPAD // NS
    mesh = plsc.VectorSubcoreMesh(core_axis_name="c", subcore_axis_name="s")

    @functools.partial(
        pl.kernel, mesh=mesh,
        compiler_params=pltpu.CompilerParams(needs_layout_passes=False, use_tc_tiling_on_sc=False),
        out_type=jax.ShapeDtypeStruct((NC, NPAD, HID), jnp.float32),
        scratch_types=[
            pltpu.VMEM((NCH, CH), jnp.int32),
            pltpu.VMEM((CH // 2, 2 * HID), jnp.float32),
            pltpu.VMEM((CH // 2, 2 * HID), jnp.float32),
            pltpu.VMEM((CH, HID), jnp.float32),
            pltpu.VMEM((CH, HID), jnp.float32),
            pltpu.VMEM_SHARED((NPAD, HID), jnp.float32),
            pltpu.SemaphoreType.DMA,
            pltpu.SemaphoreType.DMA,
            pltpu.SemaphoreType.DMA,
            pltpu.SemaphoreType.DMA,
        ],
    )
    def scatter_kernel(m2_hbm, row_hbm, zeros_hbm, out_hbm, rv, sb0, sb1,
                       mb0, mb1, acc, sems0, sems1, sema0, sema1):
        c = lax.axis_index("c")
        s = lax.axis_index("s")
        wid = s * NC + c
        sbs, mbs = [sb0, sb1], [mb0, mb1]
        semss, semas = [sems0, sems1], [sema0, sema1]
        pltpu.sync_copy(row_hbm.at[wid], rv)
        pltpu.async_copy(m2_hbm.at[pl.ds(wid * EW // 2, CH // 2)], sb0, sems0)
        pltpu.sync_copy(zeros_hbm.at[pl.ds(s * NRT, NRT)],
                        acc.at[pl.ds(s * NRT, NRT)])
        plsc.subcore_barrier()

        def chunk(j, carry):
            def body(b):
                nb = 1 - b

                @pl.when(j < NCH - 1)
                def _issue():
                    pltpu.async_copy(
                        m2_hbm.at[pl.ds((wid * EW + (j + 1) * CH) // 2,
                                        CH // 2)],
                        sbs[nb], semss[nb])

                pltpu.make_async_copy(
                    m2_hbm.at[pl.ds(wid * EW // 2, CH // 2)],
                    sbs[b], semss[b]).wait()

                @pl.when(j >= 2)
                def _drain():
                    pltpu.make_async_copy(
                        mbs[b], acc.at[rv.at[j]], semas[b]).wait()

                @plsc.parallel_loop(0, CH // 2, unroll=2)
                def repack(i2):
                    for half in range(2):
                        for t in range(HID // L):
                            mbs[b][2 * i2 + half, pl.ds(t * L, L)] = (
                                sbs[b][i2, pl.ds(half * HID + t * L, L)])
                pltpu.async_copy(mbs[b], acc.at[rv.at[j]], semas[b],
                                 add=True)

            @pl.when(j % 2 == 0)
            def _b0():
                body(0)

            @pl.when(j % 2 == 1)
            def _b1():
                body(1)

            return carry

        lax.fori_loop(0, NCH, chunk, 0)
        for b in range(2):
            pltpu.make_async_copy(mbs[b], acc.at[rv.at[0]], semas[b]).wait()
        plsc.subcore_barrier()
        pltpu.sync_copy(acc.at[pl.ds(s * NRT, NRT)],
                        out_hbm.at[c, pl.ds(s * NRT, NRT)])

    return scatter_kernel


def kernel(vel_norms, x, edge_index, charges, params):
    N = vel_norms.shape[0]
    E = edge_index.shape[1]
    LAYERS = 4

    f32 = jnp.float32

    w0a = params['node0_W'][:-3]
    w0x = params['node0_W'][-3:]
    wv = params['venc_W'] @ w0a
    b0 = (params['venc_b'] @ w0a + params['node0_b'])[None, :]
    wvx = jnp.concatenate([wv, w0x], axis=0)

    w1a = [params['edge%d_W1' % i][:HID] for i in range(LAYERS)]
    w1b = [params['edge%d_W1' % i][HID:2 * HID] for i in range(LAYERS)]
    wea = [params['edge%d_W1' % i][2 * HID] for i in range(LAYERS)]
    b1 = [params['edge%d_b1' % i][None, :] for i in range(LAYERS)]
    w2 = [params['edge%d_W2' % i] for i in range(LAYERS)]
    b2 = [params['edge%d_b2' % i][None, :] for i in range(LAYERS)]
    mw1a = [params['msg%d_W1' % i][:HID] for i in range(LAYERS)]
    mw1b = [params['msg%d_W1' % i][HID:] for i in range(LAYERS)]
    mb1 = [params['msg%d_b1' % i][None, :] for i in range(LAYERS)]
    mw2 = [params['msg%d_W2' % i] for i in range(LAYERS)]
    mb2 = [params['msg%d_b2' % i][None, :] for i in range(LAYERS)]

    OPAD = 128
    ow2p = jnp.zeros((HID, OPAD), f32).at[:, :3].set(params['out_W2'])
    ob2p = jnp.zeros((1, OPAD), f32).at[:, :3].set(params['out_b2'])
    ob1 = params['out_b1'][None, :]

    NCH = E // (NW * CH)
    row2 = edge_index[0].reshape(NW, NCH, CH)
    col2 = edge_index[1].reshape(NW, NCH, CH)
    NPAD = ((N + 8 * NS - 1) // (8 * NS)) * (8 * NS)
    zeros_n = jnp.zeros((NPAD, HID), f32)
    vx = jnp.concatenate([vel_norms, x], axis=1)

    def tc_call(body, out_shape, *args):
        return pl.pallas_call(body, out_shape=out_shape)(*args)

    nspec = jax.ShapeDtypeStruct((NPAD, HID), f32)
    h, p_tab, q_tab = tc_call(
        _node0_body, (nspec, nspec, nspec),
        vx, wvx, b0, w1a[0], w1b[0], b1[0])

    ea_kernel = _make_ea_kernel(E, N)
    gather_kernel = _make_gather_kernel(NCH, N, NPAD)
    scatter_kernel = _make_scatter_kernel(NCH, N, NPAD)

    ea2 = ea_kernel(charges, row2, col2)

    H2 = 2 * HID
    w2d = [jnp.zeros((H2, H2), f32).at[:HID, :HID].set(w)
           .at[HID:, HID:].set(w) for w in w2]
    b2d = [jnp.concatenate([b, b], axis=1) for b in b2]

    def make_edge_call(nch):
        rows = nch * CH * NW // 2
        eb = rows // 8
        return pl.pallas_call(
            _edge_body,
            grid=(rows // eb,),
            in_specs=[
                pl.BlockSpec((eb, H2), lambda i: (i, 0)),
                pl.BlockSpec((H2, H2), lambda i: (0, 0)),
                pl.BlockSpec((1, H2), lambda i: (0, 0)),
            ],
            out_specs=pl.BlockSpec((eb, H2), lambda i: (i, 0)),
            out_shape=jax.ShapeDtypeStruct((rows, H2), f32),
        )

    edge_call = make_edge_call(NCH)

    for i in range(LAYERS):
        m1 = gather_kernel(p_tab, q_tab, row2, col2, ea2, wea[i])
        m2 = edge_call(m1, w2d[i], b2d[i])
        mi = scatter_kernel(m2, row2, zeros_n)
        if i < LAYERS - 1:
            h, p_tab, q_tab = tc_call(
                _node_body, (nspec, nspec, nspec),
                h, mi, mw1a[i], mw1b[i], mb1[i], mw2[i], mb2[i],
                w1a[i + 1], w1b[i + 1], b1[i + 1])
        else:
            pred = tc_call(
                _final_body, jax.ShapeDtypeStruct((NPAD, OPAD), f32),
                h, mi, mw1a[i], mw1b[i], mb1[i], mw2[i], mb2[i],
                params['out_W1'], ob1, ow2p, ob2p)

    return pred[:N, :3]
